# initial kernel scaffold (unmeasured)
import jax
import jax.numpy as jnp
from jax import lax
from jax.experimental import pallas as pl
from jax.experimental.pallas import tpu as pltpu


def kernel(
    x,
):
    def body(*refs):
        pass

    out_shape = jax.ShapeDtypeStruct(..., jnp.float32)
    return pl.pallas_call(body, out_shape=out_shape)(...)


# baseline (device time: 35152 ns/iter reference)
import jax
import jax.numpy as jnp
from jax import lax
from jax.experimental import pallas as pl
from jax.experimental.pallas import tpu as pltpu

M = 512
N = 512


def kernel(x):
    x = x.reshape(M, N)

    def body(x_ref, out_ref, recv_buf, send_sems, recv_sems):
        cx = lax.axis_index("x")
        cy = lax.axis_index("y")
        cz = lax.axis_index("z")

        partner_x = (1 - cx, cy, cz)
        partner_y = (cx, 1 - cy, cz)
        partner_z = (cx, cy, 1 - cz)

        barrier = pltpu.get_barrier_semaphore()
        for p in (partner_x, partner_y, partner_z):
            pl.semaphore_signal(
                barrier, inc=1, device_id=p,
                device_id_type=pl.DeviceIdType.MESH,
            )
        pl.semaphore_wait(barrier, 3)

        out_ref[...] = x_ref[...]

        base_z = cz * 256
        base_y = base_z + cy * 128
        base_x = base_y + cx * 64

        rs_steps = [
            (0, partner_z, (1 - cz) * 256, base_z, 256, 0),
            (1, partner_y, base_z + (1 - cy) * 128, base_y, 128, 256),
            (2, partner_x, base_y + (1 - cx) * 64, base_x, 64, 384),
        ]
        for step, pdev, send_off, keep_off, rows, roff in rs_steps:
            rdma = pltpu.make_async_remote_copy(
                src_ref=out_ref.at[pl.ds(send_off, rows), :],
                dst_ref=recv_buf.at[pl.ds(roff, rows), :],
                send_sem=send_sems.at[step],
                recv_sem=recv_sems.at[step],
                device_id=pdev,
                device_id_type=pl.DeviceIdType.MESH,
            )
            rdma.start()
            rdma.wait()
            out_ref[pl.ds(keep_off, rows), :] = (
                out_ref[pl.ds(keep_off, rows), :]
                + recv_buf[pl.ds(roff, rows), :]
            )

        ag_steps = [
            (3, partner_x, base_x, 64),
            (4, partner_y, base_y, 128),
            (5, partner_z, base_z, 256),
        ]
        for step, pdev, off, rows in ag_steps:
            rdma = pltpu.make_async_remote_copy(
                src_ref=out_ref.at[pl.ds(off, rows), :],
                dst_ref=out_ref.at[pl.ds(off, rows), :],
                send_sem=send_sems.at[step],
                recv_sem=recv_sems.at[step],
                device_id=pdev,
                device_id_type=pl.DeviceIdType.MESH,
            )
            rdma.start()
            rdma.wait()

    return pl.pallas_call(
        body,
        out_shape=jax.ShapeDtypeStruct((M, N), jnp.float32),
        in_specs=[pl.BlockSpec(memory_space=pltpu.VMEM)],
        out_specs=pl.BlockSpec(memory_space=pltpu.VMEM),
        scratch_shapes=[
            pltpu.VMEM((448, N), jnp.float32),
            pltpu.SemaphoreType.DMA((6,)),
            pltpu.SemaphoreType.DMA((6,)),
        ],
        compiler_params=pltpu.CompilerParams(collective_id=0),
    )(x)


# device time: 24215 ns/iter; 1.4517x vs baseline; 1.4517x over previous
import jax
import jax.numpy as jnp
from jax import lax
from jax.experimental import pallas as pl
from jax.experimental.pallas import tpu as pltpu

M = 512
N = 512
SEG = 64

_PEER_BITS = [
    (1, 0, 0), (0, 1, 0), (0, 0, 1),
    (1, 1, 0), (1, 0, 1), (0, 1, 1),
    (1, 1, 1),
]


def kernel(x):
    x = x.reshape(M, N)

    def body(x_ref, out_ref, rs_buf, rs_send, rs_recv, ag_send, ag_recv):
        cx = lax.axis_index("x")
        cy = lax.axis_index("y")
        cz = lax.axis_index("z")
        me = cz * 4 + cy * 2 + cx

        def peer(bits):
            bx, by, bz = bits
            px = 1 - cx if bx else cx
            py = 1 - cy if by else cy
            pz = 1 - cz if bz else cz
            return (px, py, pz), pz * 4 + py * 2 + px

        barrier = pltpu.get_barrier_semaphore()
        for bits in _PEER_BITS:
            p, _ = peer(bits)
            pl.semaphore_signal(
                barrier, inc=1, device_id=p,
                device_id_type=pl.DeviceIdType.MESH,
            )
        pl.semaphore_wait(barrier, 7)

        for bits in _PEER_BITS:
            p, ps = peer(bits)
            rdma = pltpu.make_async_remote_copy(
                src_ref=x_ref.at[pl.ds(ps * SEG, SEG), :],
                dst_ref=rs_buf.at[me],
                send_sem=rs_send.at[ps],
                recv_sem=rs_recv.at[me],
                device_id=p,
                device_id_type=pl.DeviceIdType.MESH,
            )
            rdma.start()

        acc = x_ref[pl.ds(me * SEG, SEG), :]
        for bits in _PEER_BITS:
            p, ps = peer(bits)
            recv = pltpu.make_async_remote_copy(
                src_ref=rs_buf.at[ps],
                dst_ref=rs_buf.at[ps],
                send_sem=rs_send.at[ps],
                recv_sem=rs_recv.at[ps],
                device_id=p,
                device_id_type=pl.DeviceIdType.MESH,
            )
            recv.wait_recv()
            acc = acc + rs_buf[ps]
        out_ref[pl.ds(me * SEG, SEG), :] = acc

        for bits in _PEER_BITS:
            p, ps = peer(bits)
            rdma = pltpu.make_async_remote_copy(
                src_ref=out_ref.at[pl.ds(me * SEG, SEG), :],
                dst_ref=out_ref.at[pl.ds(me * SEG, SEG), :],
                send_sem=ag_send.at[ps],
                recv_sem=ag_recv.at[me],
                device_id=p,
                device_id_type=pl.DeviceIdType.MESH,
            )
            rdma.start()

        for bits in _PEER_BITS:
            p, ps = peer(bits)
            recv = pltpu.make_async_remote_copy(
                src_ref=out_ref.at[pl.ds(ps * SEG, SEG), :],
                dst_ref=out_ref.at[pl.ds(ps * SEG, SEG), :],
                send_sem=ag_send.at[ps],
                recv_sem=ag_recv.at[ps],
                device_id=p,
                device_id_type=pl.DeviceIdType.MESH,
            )
            recv.wait_recv()

        for bits in _PEER_BITS:
            p, ps = peer(bits)
            for sem in (rs_send, ag_send):
                drain = pltpu.make_async_remote_copy(
                    src_ref=rs_buf.at[ps],
                    dst_ref=rs_buf.at[ps],
                    send_sem=sem.at[ps],
                    recv_sem=rs_recv.at[ps],
                    device_id=p,
                    device_id_type=pl.DeviceIdType.MESH,
                )
                drain.wait_send()

    return pl.pallas_call(
        body,
        out_shape=jax.ShapeDtypeStruct((M, N), jnp.float32),
        in_specs=[pl.BlockSpec(memory_space=pltpu.VMEM)],
        out_specs=pl.BlockSpec(memory_space=pltpu.VMEM),
        scratch_shapes=[
            pltpu.VMEM((8, SEG, N), jnp.float32),
            pltpu.SemaphoreType.DMA((8,)),
            pltpu.SemaphoreType.DMA((8,)),
            pltpu.SemaphoreType.DMA((8,)),
            pltpu.SemaphoreType.DMA((8,)),
        ],
        compiler_params=pltpu.CompilerParams(collective_id=0),
    )(x)


# device time: 20268 ns/iter; 1.7344x vs baseline; 1.1947x over previous
import jax
import jax.numpy as jnp
from jax import lax
from jax.experimental import pallas as pl
from jax.experimental.pallas import tpu as pltpu

M = 512
N = 512
SEG = 64
H = 2
NH = N // H

WAIT_ORDER = [
    (1, 0, 0), (0, 1, 0), (0, 0, 1),
    (1, 1, 0), (1, 0, 1), (0, 1, 1),
    (1, 1, 1),
]
ISSUE_ORDER = [
    (1, 1, 1), (0, 1, 1), (1, 0, 1), (1, 1, 0),
    (0, 0, 1), (0, 1, 0), (1, 0, 0),
]


def kernel(x):
    x = x.reshape(M, N)

    def body(x_ref, out_ref, rs_buf, rs_send, rs_recv, ag_send, ag_recv):
        cx = lax.axis_index("x")
        cy = lax.axis_index("y")
        cz = lax.axis_index("z")
        me = cz * 4 + cy * 2 + cx

        def peer(bits):
            bx, by, bz = bits
            px = 1 - cx if bx else cx
            py = 1 - cy if by else cy
            pz = 1 - cz if bz else cz
            return (px, py, pz), pz * 4 + py * 2 + px

        barrier = pltpu.get_barrier_semaphore()
        for bits in WAIT_ORDER:
            p, _ = peer(bits)
            pl.semaphore_signal(
                barrier, inc=1, device_id=p,
                device_id_type=pl.DeviceIdType.MESH,
            )
        pl.semaphore_wait(barrier, 7)

        for h in range(H):
            for bits in ISSUE_ORDER:
                p, ps = peer(bits)
                pltpu.make_async_remote_copy(
                    src_ref=x_ref.at[pl.ds(ps * SEG, SEG), pl.ds(h * NH, NH)],
                    dst_ref=rs_buf.at[h, me],
                    send_sem=rs_send.at[h * 8 + ps],
                    recv_sem=rs_recv.at[h * 8 + me],
                    device_id=p,
                    device_id_type=pl.DeviceIdType.MESH,
                ).start()

        for h in range(H):
            acc = x_ref[pl.ds(me * SEG, SEG), pl.ds(h * NH, NH)]
            for bits in WAIT_ORDER:
                p, ps = peer(bits)
                recv = pltpu.make_async_remote_copy(
                    src_ref=rs_buf.at[h, ps],
                    dst_ref=rs_buf.at[h, ps],
                    send_sem=rs_send.at[h * 8 + ps],
                    recv_sem=rs_recv.at[h * 8 + ps],
                    device_id=p,
                    device_id_type=pl.DeviceIdType.MESH,
                )
                recv.wait_recv()
                acc = acc + rs_buf[h, ps]
            out_ref[pl.ds(me * SEG, SEG), pl.ds(h * NH, NH)] = acc
            for bits in ISSUE_ORDER:
                p, ps = peer(bits)
                pltpu.make_async_remote_copy(
                    src_ref=out_ref.at[pl.ds(me * SEG, SEG),
                                       pl.ds(h * NH, NH)],
                    dst_ref=out_ref.at[pl.ds(me * SEG, SEG),
                                       pl.ds(h * NH, NH)],
                    send_sem=ag_send.at[h * 8 + ps],
                    recv_sem=ag_recv.at[h * 8 + me],
                    device_id=p,
                    device_id_type=pl.DeviceIdType.MESH,
                ).start()

        for h in range(H):
            for bits in WAIT_ORDER:
                p, ps = peer(bits)
                pltpu.make_async_remote_copy(
                    src_ref=out_ref.at[pl.ds(ps * SEG, SEG),
                                       pl.ds(h * NH, NH)],
                    dst_ref=out_ref.at[pl.ds(ps * SEG, SEG),
                                       pl.ds(h * NH, NH)],
                    send_sem=ag_send.at[h * 8 + ps],
                    recv_sem=ag_recv.at[h * 8 + ps],
                    device_id=p,
                    device_id_type=pl.DeviceIdType.MESH,
                ).wait_recv()

        for h in range(H):
            for bits in WAIT_ORDER:
                p, ps = peer(bits)
                for sem in (rs_send, ag_send):
                    pltpu.make_async_remote_copy(
                        src_ref=rs_buf.at[h, ps],
                        dst_ref=rs_buf.at[h, ps],
                        send_sem=sem.at[h * 8 + ps],
                        recv_sem=rs_recv.at[h * 8 + ps],
                        device_id=p,
                        device_id_type=pl.DeviceIdType.MESH,
                    ).wait_send()

    return pl.pallas_call(
        body,
        out_shape=jax.ShapeDtypeStruct((M, N), jnp.float32),
        in_specs=[pl.BlockSpec(memory_space=pltpu.VMEM)],
        out_specs=pl.BlockSpec(memory_space=pltpu.VMEM),
        scratch_shapes=[
            pltpu.VMEM((H, 8, SEG, NH), jnp.float32),
            pltpu.SemaphoreType.DMA((H * 8,)),
            pltpu.SemaphoreType.DMA((H * 8,)),
            pltpu.SemaphoreType.DMA((H * 8,)),
            pltpu.SemaphoreType.DMA((H * 8,)),
        ],
        compiler_params=pltpu.CompilerParams(collective_id=0),
    )(x)
